# baseline (speedup 1.0000x reference)
import functools

import jax
import jax.numpy as jnp
from jax import lax
from jax.experimental import pallas as pl
from jax.experimental.pallas import tpu as pltpu
from jax.experimental.pallas import tpu_sc as plsc

NUM_RANKS = 13
NUM_SUITS = 4
EMBED_DIM = 128
BATCH = 16384
HIST = 200

_NC = 2
_NS = 16
_NW = _NC * _NS
_ROWS = BATCH * HIST
_ROWS_PER_W = _ROWS // _NW
_UNIT = 128
_NUNIT = _ROWS_PER_W // _UNIT
_NBUF = 4
_RTAB = NUM_RANKS + 1
_STAB = NUM_SUITS + 1
_CTAB = _RTAB * _STAB


def _vgather(vec, idx):
  dnums = lax.GatherDimensionNumbers(
      offset_dims=(), collapsed_slice_dims=(0,), start_index_map=(0,))
  return lax.gather(vec, idx[:, None], dnums, (1,),
                    mode=lax.GatherScatterMode.PROMISE_IN_BOUNDS)


def _body(x_hbm, rank_hbm, suit_hbm, out_hbm,
          rank_v, suit_v, tab, tab_sh, xvs, cidxs, obs,
          xsems, gsems, osems):
  wid = lax.axis_index("s") * _NC + lax.axis_index("c")
  row0 = wid * _ROWS_PER_W

  pltpu.sync_copy(rank_hbm, rank_v)
  pltpu.sync_copy(suit_hbm, suit_v)

  def build_r(r, _):
    def build_s(s, _):
      c = r * _STAB + s
      for k in range(EMBED_DIM // 16):
        sl = pl.ds(k * 16, 16)
        tab[c, sl] = rank_v[r, sl] + suit_v[s, sl]
      return 0
    return lax.fori_loop(0, _STAB, build_s, 0)
  lax.fori_loop(0, _RTAB, build_r, 0)

  @pl.when(lax.axis_index("s") == 0)
  def _():
    pltpu.sync_copy(tab, tab_sh)
  plsc.subcore_barrier()

  def x_copies(unit, b):
    base = (row0 + unit * _UNIT) * 2
    return (
        pltpu.make_async_copy(
            x_hbm.at[pl.ds(base, 2 * _UNIT)], xvs[b], xsems[b]),
    )

  def gather_copy(b):
    return pltpu.make_async_copy(tab_sh.at[cidxs[b]], obs[b], gsems[b])

  def out_copy(unit, b):
    base = row0 + unit * _UNIT
    return pltpu.make_async_copy(
        obs[b], out_hbm.at[pl.ds(base, _UNIT)], osems[b])

  for c in x_copies(0, 0) + x_copies(1, 1):
    c.start()

  def do_unit(u, b):
    bp = (b - 1) % _NBUF

    for c in x_copies(u, b):
      c.wait()
    nxt = lax.rem(u + 2, _NUNIT)
    for c in x_copies(nxt, (b + 2) % _NBUF):
      c.start()

    iota = lax.iota(jnp.int32, 16)
    perm = (iota * 2) % 16
    lo = iota < 8
    for v in range(_UNIT // 16):
      a = xvs[b][pl.ds(v * 32, 16)]
      c = xvs[b][pl.ds(v * 32 + 16, 16)]
      r = jnp.where(lo, _vgather(a, perm), _vgather(c, perm))
      s = jnp.where(lo, _vgather(a, perm + 1), _vgather(c, perm + 1))
      cidxs[b][pl.ds(v * 16, 16)] = r * _STAB + s

    @pl.when(u >= _NBUF)
    def _():
      out_copy(u - _NBUF, b).wait()

    gather_copy(b).start()

    bpp = (b - 2) % _NBUF
    @pl.when(u >= 2)
    def _():
      gather_copy(bpp).wait()
      out_copy(u - 2, bpp).start()

  def loop(g, _):
    for i in range(_NBUF):
      do_unit(g * _NBUF + i, i)
    return 0
  lax.fori_loop(0, _NUNIT // _NBUF, loop, 0)

  for u in (_NUNIT - 2, _NUNIT - 1):
    gather_copy(u % _NBUF).wait()
    out_copy(u, u % _NBUF).start()
  for u in range(_NUNIT - _NBUF, _NUNIT):
    out_copy(u, u % _NBUF).wait()
  for c in x_copies(0, 0) + x_copies(1, 1):
    c.wait()


@jax.jit
def _run(x_flat, rank_table, suit_table):
  mesh = plsc.VectorSubcoreMesh(core_axis_name="c", subcore_axis_name="s")
  f = functools.partial(
      pl.kernel,
      out_type=jax.ShapeDtypeStruct((_ROWS, EMBED_DIM), jnp.float32),
      mesh=mesh,
      scratch_types=[
          pltpu.VMEM((_RTAB, EMBED_DIM), jnp.float32),
          pltpu.VMEM((_STAB, EMBED_DIM), jnp.float32),
          pltpu.VMEM((_CTAB, EMBED_DIM), jnp.float32),
          pltpu.VMEM_SHARED((_CTAB, EMBED_DIM), jnp.float32),
          [pltpu.VMEM((2 * _UNIT,), jnp.int32) for _ in range(_NBUF)],
          [pltpu.VMEM((_UNIT,), jnp.int32) for _ in range(_NBUF)],
          [pltpu.VMEM((_UNIT, EMBED_DIM), jnp.float32) for _ in range(_NBUF)],
          [pltpu.SemaphoreType.DMA for _ in range(_NBUF)],
          [pltpu.SemaphoreType.DMA for _ in range(_NBUF)],
          [pltpu.SemaphoreType.DMA for _ in range(_NBUF)],
      ],
  )(_body)
  return f(x_flat, rank_table, suit_table)


def kernel(x, rank_table, suit_table):
  out = _run(x.reshape(-1), rank_table, suit_table)
  return out.reshape(BATCH, HIST, EMBED_DIM)


# speedup vs baseline: 3.9894x; 3.9894x over previous
import functools

import jax
import jax.numpy as jnp
from jax import lax
from jax.experimental import pallas as pl
from jax.experimental.pallas import tpu as pltpu
from jax.experimental.pallas import tpu_sc as plsc

NUM_RANKS = 13
NUM_SUITS = 4
EMBED_DIM = 128
BATCH = 16384
HIST = 200

_NC = 2
_NS = 16
_NW = _NC * _NS
_ROWS = BATCH * HIST
_ROWS_PER_W = _ROWS // _NW
_UNIT = 128
_NUNIT = _ROWS_PER_W // _UNIT
_NBUF = 4
_RTAB = NUM_RANKS + 1
_STAB = NUM_SUITS + 1
_CTAB = _RTAB * _STAB


def _body(xt_hbm, rank_hbm, suit_hbm, out_hbm,
          rank_v, suit_v, tab, tab_sh, xvs, cidxs, obs,
          xsems, gsems, osems):
  wid = lax.axis_index("s") * _NC + lax.axis_index("c")
  row0 = wid * _ROWS_PER_W

  pltpu.sync_copy(rank_hbm, rank_v)
  pltpu.sync_copy(suit_hbm, suit_v)

  def build_r(r, _):
    def build_s(s, _):
      c = r * _STAB + s
      for k in range(EMBED_DIM // 16):
        sl = pl.ds(k * 16, 16)
        tab[c, sl] = rank_v[r, sl] + suit_v[s, sl]
      return 0
    return lax.fori_loop(0, _STAB, build_s, 0)
  lax.fori_loop(0, _RTAB, build_r, 0)

  @pl.when(lax.axis_index("s") == 0)
  def _():
    pltpu.sync_copy(tab, tab_sh)
  plsc.subcore_barrier()

  def x_copies(unit, b):
    base = row0 + unit * _UNIT
    return (
        pltpu.make_async_copy(
            xt_hbm.at[0, pl.ds(base, _UNIT)], xvs[b].at[0], xsems[b]),
        pltpu.make_async_copy(
            xt_hbm.at[1, pl.ds(base, _UNIT)], xvs[b].at[1], xsems[b]),
    )

  def gather_copy(b):
    return pltpu.make_async_copy(tab_sh.at[cidxs[b]], obs[b], gsems[b])

  def out_copy(unit, b):
    base = row0 + unit * _UNIT
    return pltpu.make_async_copy(
        obs[b], out_hbm.at[pl.ds(base, _UNIT)], osems[b])

  for c in x_copies(0, 0) + x_copies(1, 1):
    c.start()

  def do_unit(u, b):
    bp = (b - 1) % _NBUF

    for c in x_copies(u, b):
      c.wait()
    nxt = lax.rem(u + 2, _NUNIT)
    for c in x_copies(nxt, (b + 2) % _NBUF):
      c.start()

    for v in range(_UNIT // 16):
      sl = pl.ds(v * 16, 16)
      cidxs[b][sl] = xvs[b][0, sl] * _STAB + xvs[b][1, sl]

    @pl.when(u >= _NBUF)
    def _():
      out_copy(u - _NBUF, b).wait()

    gather_copy(b).start()

    bpp = (b - 2) % _NBUF
    @pl.when(u >= 2)
    def _():
      gather_copy(bpp).wait()
      out_copy(u - 2, bpp).start()

  def loop(g, _):
    for i in range(_NBUF):
      do_unit(g * _NBUF + i, i)
    return 0
  lax.fori_loop(0, _NUNIT // _NBUF, loop, 0)

  for u in (_NUNIT - 2, _NUNIT - 1):
    gather_copy(u % _NBUF).wait()
    out_copy(u, u % _NBUF).start()
  for u in range(_NUNIT - _NBUF, _NUNIT):
    out_copy(u, u % _NBUF).wait()
  for c in x_copies(0, 0) + x_copies(1, 1):
    c.wait()


@jax.jit
def _run(xt, rank_table, suit_table):
  mesh = plsc.VectorSubcoreMesh(core_axis_name="c", subcore_axis_name="s")
  f = functools.partial(
      pl.kernel,
      out_type=jax.ShapeDtypeStruct((_ROWS, EMBED_DIM), jnp.float32),
      mesh=mesh,
      scratch_types=[
          pltpu.VMEM((_RTAB, EMBED_DIM), jnp.float32),
          pltpu.VMEM((_STAB, EMBED_DIM), jnp.float32),
          pltpu.VMEM((_CTAB, EMBED_DIM), jnp.float32),
          pltpu.VMEM_SHARED((_CTAB, EMBED_DIM), jnp.float32),
          [pltpu.VMEM((2, _UNIT), jnp.int32) for _ in range(_NBUF)],
          [pltpu.VMEM((_UNIT,), jnp.int32) for _ in range(_NBUF)],
          [pltpu.VMEM((_UNIT, EMBED_DIM), jnp.float32) for _ in range(_NBUF)],
          [pltpu.SemaphoreType.DMA for _ in range(_NBUF)],
          [pltpu.SemaphoreType.DMA for _ in range(_NBUF)],
          [pltpu.SemaphoreType.DMA for _ in range(_NBUF)],
      ],
  )(_body)
  return f(xt, rank_table, suit_table)


def kernel(x, rank_table, suit_table):
  xt = x.reshape(-1, 2).T
  out = _run(xt, rank_table, suit_table)
  return out.reshape(BATCH, HIST, EMBED_DIM)


# speedup vs baseline: 6.9328x; 1.7378x over previous
import functools

import jax
import jax.numpy as jnp
from jax import lax
from jax.experimental import pallas as pl
from jax.experimental.pallas import tpu as pltpu
from jax.experimental.pallas import tpu_sc as plsc

NUM_RANKS = 13
NUM_SUITS = 4
EMBED_DIM = 128
BATCH = 16384
HIST = 200

_NC = 2
_NS = 16
_NW = _NC * _NS
_ROWS = BATCH * HIST
_ROWS_PER_W = _ROWS // _NW
_UNIT = 128
_NUNIT = _ROWS_PER_W // _UNIT
_NBUF = 4
_RTAB = NUM_RANKS + 1
_STAB = NUM_SUITS + 1
_CTAB = _RTAB * _STAB


def _body(cmb_hbm, rank_hbm, suit_hbm, out_hbm,
          rank_v, suit_v, tab, tab_sh, cidxs, obs,
          xsems, gsems, osems):
  wid = lax.axis_index("s") * _NC + lax.axis_index("c")
  row0 = wid * _ROWS_PER_W

  pltpu.sync_copy(rank_hbm, rank_v)
  pltpu.sync_copy(suit_hbm, suit_v)

  def build_r(r, _):
    def build_s(s, _):
      c = r * _STAB + s
      for k in range(EMBED_DIM // 16):
        sl = pl.ds(k * 16, 16)
        tab[c, sl] = rank_v[r, sl] + suit_v[s, sl]
      return 0
    return lax.fori_loop(0, _STAB, build_s, 0)
  lax.fori_loop(0, _RTAB, build_r, 0)

  @pl.when(lax.axis_index("s") == 0)
  def _():
    pltpu.sync_copy(tab, tab_sh)
  plsc.subcore_barrier()

  def x_copy(unit, b):
    base = row0 + unit * _UNIT
    return pltpu.make_async_copy(
        cmb_hbm.at[pl.ds(base, _UNIT)], cidxs[b], xsems[b])

  def gather_copy(b):
    return pltpu.make_async_copy(tab_sh.at[cidxs[b]], obs[b], gsems[b])

  def out_copy(unit, b):
    base = row0 + unit * _UNIT
    return pltpu.make_async_copy(
        obs[b], out_hbm.at[pl.ds(base, _UNIT)], osems[b])

  x_copy(0, 0).start()
  x_copy(1, 1).start()

  def do_unit(u, b):
    x_copy(u, b).wait()

    @pl.when(u >= _NBUF)
    def _():
      out_copy(u - _NBUF, b).wait()

    gather_copy(b).start()

    bpp = (b - 2) % _NBUF
    @pl.when(u >= 2)
    def _():
      gather_copy(bpp).wait()
      out_copy(u - 2, bpp).start()
    nxt = lax.rem(u + 2, _NUNIT)
    x_copy(nxt, (b + 2) % _NBUF).start()

  def loop(g, _):
    for i in range(_NBUF):
      do_unit(g * _NBUF + i, i)
    return 0
  lax.fori_loop(0, _NUNIT // _NBUF, loop, 0)

  for u in (_NUNIT - 2, _NUNIT - 1):
    gather_copy(u % _NBUF).wait()
    out_copy(u, u % _NBUF).start()
  for u in range(_NUNIT - _NBUF, _NUNIT):
    out_copy(u, u % _NBUF).wait()
  x_copy(0, 0).wait()
  x_copy(1, 1).wait()


@jax.jit
def _run(cmb, rank_table, suit_table):
  mesh = plsc.VectorSubcoreMesh(core_axis_name="c", subcore_axis_name="s")
  f = functools.partial(
      pl.kernel,
      out_type=jax.ShapeDtypeStruct((_ROWS, EMBED_DIM), jnp.float32),
      mesh=mesh,
      scratch_types=[
          pltpu.VMEM((_RTAB, EMBED_DIM), jnp.float32),
          pltpu.VMEM((_STAB, EMBED_DIM), jnp.float32),
          pltpu.VMEM((_CTAB, EMBED_DIM), jnp.float32),
          pltpu.VMEM_SHARED((_CTAB, EMBED_DIM), jnp.float32),
          [pltpu.VMEM((_UNIT,), jnp.int32) for _ in range(_NBUF)],
          [pltpu.VMEM((_UNIT, EMBED_DIM), jnp.float32) for _ in range(_NBUF)],
          [pltpu.SemaphoreType.DMA for _ in range(_NBUF)],
          [pltpu.SemaphoreType.DMA for _ in range(_NBUF)],
          [pltpu.SemaphoreType.DMA for _ in range(_NBUF)],
      ],
  )(_body)
  return f(cmb, rank_table, suit_table)


def kernel(x, rank_table, suit_table):
  cmb = (x[..., 0] * _STAB + x[..., 1]).reshape(-1)
  out = _run(cmb, rank_table, suit_table)
  return out.reshape(BATCH, HIST, EMBED_DIM)
